# Initial kernel scaffold; baseline (speedup 1.0000x reference)
import jax
import jax.numpy as jnp
from jax.experimental import pallas as pl


def kernel(x, edge_index, edge_weight, W1, b1, W2, b2):
    raise NotImplementedError("write your pallas kernel here")


# speedup vs baseline: 5.2278x; 5.2278x over previous
import functools

import jax
import jax.numpy as jnp
from jax import lax
from jax.experimental import pallas as pl
from jax.experimental.pallas import tpu as pltpu
from jax.experimental.pallas import tpu_sc as plsc

N_NODES = 10000
N_EDGES = 320000
NFEAT = 128
NCLASS = 40

NC = 2
NS = 16
NW = NC * NS
EPT = N_EDGES // NW
B = 80
NITER = EPT // B
NPAD = 10240
ZR = NPAD // NS


_GDN = lax.GatherDimensionNumbers(
    offset_dims=(), collapsed_slice_dims=(0,), start_index_map=(0,))


def _build_spmm(D):
    mesh = plsc.VectorSubcoreMesh(core_axis_name="c", subcore_axis_name="s")

    @functools.partial(
        pl.kernel,
        out_type=jax.ShapeDtypeStruct((NC, NPAD, D), jnp.float32),
        mesh=mesh,
        scratch_types=[
            pltpu.VMEM((NITER, B), jnp.int32),
            pltpu.VMEM((NITER, B), jnp.int32),
            pltpu.VMEM((B,), jnp.float32),
            pltpu.VMEM((B, D), jnp.float32),
            pltpu.VMEM_SHARED((NPAD, D), jnp.float32),
            pltpu.SemaphoreType.DMA,
        ],
    )
    def spmm(src_hbm, dst_hbm, w_hbm, table_hbm, zeros_hbm, out_hbm,
             src_v, dst_v, w_v, rows_v, accum, sem):
        c = lax.axis_index("c")
        s = lax.axis_index("s")
        wid = c * NS + s
        pltpu.sync_copy(zeros_hbm, accum.at[pl.ds(s * ZR, ZR)])
        pltpu.sync_copy(src_hbm.at[wid], src_v)
        pltpu.sync_copy(dst_hbm.at[wid], dst_v)
        plsc.subcore_barrier()

        def body(i, carry):
            pltpu.sync_copy(w_hbm.at[wid, i], w_v)
            pltpu.async_copy(table_hbm.at[src_v.at[i]], rows_v, sem).wait()
            for g in range(B // 16):
                w16 = w_v[pl.ds(g * 16, 16)]
                for j in range(16):
                    e = g * 16 + j
                    wv = lax.gather(
                        w16, jnp.full((16, 1), j, jnp.int32), _GDN,
                        slice_sizes=(1,),
                        mode=lax.GatherScatterMode.PROMISE_IN_BOUNDS)
                    for d in range(D // 16):
                        sl = (e, pl.ds(d * 16, 16))
                        rows_v[sl] = rows_v[sl] * wv
            pltpu.sync_copy(rows_v, accum.at[dst_v.at[i]], add=True)
            return carry

        lax.fori_loop(0, NITER, body, 0)
        plsc.subcore_barrier()
        pltpu.sync_copy(accum.at[pl.ds(s * ZR, ZR)],
                        out_hbm.at[c, pl.ds(s * ZR, ZR)])

    return spmm


_spmm128 = _build_spmm(NFEAT)


BM = 1000


def _dense1_body(x_ref, a_ref, b_ref, w1_ref, b1_ref, h_ref):
    t = x_ref[...] + a_ref[...] + b_ref[...]
    h = jnp.dot(t, w1_ref[...], preferred_element_type=jnp.float32)
    h_ref[...] = jnp.maximum(h + b1_ref[...], 0.0)


def _dense1(x, s1a, s1b, W1, b1):
    return pl.pallas_call(
        _dense1_body,
        grid=(N_NODES // BM,),
        in_specs=[
            pl.BlockSpec((BM, NFEAT), lambda i: (i, 0)),
            pl.BlockSpec((BM, NFEAT), lambda i: (i, 0)),
            pl.BlockSpec((BM, NFEAT), lambda i: (i, 0)),
            pl.BlockSpec((NFEAT, NFEAT), lambda i: (0, 0)),
            pl.BlockSpec((1, NFEAT), lambda i: (0, 0)),
        ],
        out_specs=pl.BlockSpec((BM, NFEAT), lambda i: (i, 0)),
        out_shape=jax.ShapeDtypeStruct((N_NODES, NFEAT), jnp.float32),
    )(x, s1a, s1b, W1, b1)


def _dense2_body(h_ref, a_ref, b_ref, w2_ref, b2_ref, o_ref):
    t = h_ref[...] + a_ref[...] + b_ref[...]
    z = jnp.dot(t, w2_ref[...], preferred_element_type=jnp.float32)
    z = z + b2_ref[...]
    m = jnp.max(z, axis=1, keepdims=True)
    zm = z - m
    o_ref[...] = zm - jnp.log(jnp.sum(jnp.exp(zm), axis=1, keepdims=True))


def _dense2(h, s2a, s2b, W2, b2):
    return pl.pallas_call(
        _dense2_body,
        grid=(N_NODES // BM,),
        in_specs=[
            pl.BlockSpec((BM, NFEAT), lambda i: (i, 0)),
            pl.BlockSpec((BM, NFEAT), lambda i: (i, 0)),
            pl.BlockSpec((BM, NFEAT), lambda i: (i, 0)),
            pl.BlockSpec((NFEAT, NCLASS), lambda i: (0, 0)),
            pl.BlockSpec((1, NCLASS), lambda i: (0, 0)),
        ],
        out_specs=pl.BlockSpec((BM, NCLASS), lambda i: (i, 0)),
        out_shape=jax.ShapeDtypeStruct((N_NODES, NCLASS), jnp.float32),
    )(h, s2a, s2b, W2, b2)


def kernel(x, edge_index, edge_weight, W1, b1, W2, b2):
    dst = edge_index[0].reshape(NW, NITER, B)
    src = edge_index[1].reshape(NW, NITER, B)
    w3 = edge_weight.reshape(NW, NITER, B)
    zeros128 = jnp.zeros((ZR, NFEAT), jnp.float32)

    s1 = _spmm128(src, dst, w3, x, zeros128)
    h = _dense1(x, s1[0, :N_NODES], s1[1, :N_NODES], W1, b1.reshape(1, -1))
    s2 = _spmm128(src, dst, w3, h, zeros128)
    return _dense2(h, s2[0, :N_NODES], s2[1, :N_NODES], W2, b2.reshape(1, -1))


# speedup vs baseline: 5.6335x; 1.0776x over previous
import functools

import jax
import jax.numpy as jnp
from jax import lax
from jax.experimental import pallas as pl
from jax.experimental.pallas import tpu as pltpu
from jax.experimental.pallas import tpu_sc as plsc

N_NODES = 10000
N_EDGES = 320000
NFEAT = 128
NCLASS = 40

NC = 2
NS = 16
NW = NC * NS
EPT = N_EDGES // NW
B = 80
NITER = EPT // B
NWPAD = 128
NPAD = 10240
ZR = NPAD // NS

_GDN = lax.GatherDimensionNumbers(
    offset_dims=(), collapsed_slice_dims=(0,), start_index_map=(0,))


def _build_spmm(D):
    mesh = plsc.VectorSubcoreMesh(core_axis_name="c", subcore_axis_name="s")

    def scale(wb, rows):
        for g in range(B // 16):
            w16 = wb[pl.ds(g * 16, 16)]
            for j in range(16):
                wv = lax.gather(
                    w16, jnp.full((16, 1), j, jnp.int32), _GDN,
                    slice_sizes=(1,),
                    mode=lax.GatherScatterMode.PROMISE_IN_BOUNDS)
                e = g * 16 + j
                for d in range(D // 16):
                    sl = (e, pl.ds(d * 16, 16))
                    rows[sl] = rows[sl] * wv

    @functools.partial(
        pl.kernel,
        out_type=jax.ShapeDtypeStruct((NC, NPAD, D), jnp.float32),
        mesh=mesh,
        scratch_types=[
            pltpu.VMEM((2, B), jnp.int32),
            pltpu.VMEM((2, B), jnp.int32),
            pltpu.VMEM((B,), jnp.float32),
            pltpu.VMEM((B,), jnp.float32),
            pltpu.VMEM((B, D), jnp.float32),
            pltpu.VMEM((B, D), jnp.float32),
            pltpu.VMEM_SHARED((NPAD, D), jnp.float32),
            pltpu.SemaphoreType.DMA,
            pltpu.SemaphoreType.DMA,
            pltpu.SemaphoreType.DMA,
            pltpu.SemaphoreType.DMA,
            pltpu.SemaphoreType.DMA,
            pltpu.SemaphoreType.DMA,
        ],
    )
    def spmm(edges_hbm, w_hbm, table_hbm, zeros_hbm, out_hbm,
             eb0, eb1, wb0, wb1, rows0, rows1, accum,
             sem_e0, sem_e1, sem_g0, sem_g1, sem_s0, sem_s1):
        c = lax.axis_index("c")
        s = lax.axis_index("s")
        wid = c * NS + s

        def edge_copy(i, eb, sem):
            return pltpu.make_async_copy(edges_hbm.at[wid, i], eb, sem)

        def w_copy(i, wb, sem):
            return pltpu.make_async_copy(w_hbm.at[wid, i], wb, sem)

        def gather(eb, rows, sem):
            return pltpu.make_async_copy(table_hbm.at[eb.at[0]], rows, sem)

        def scatter(eb, rows, sem):
            return pltpu.make_async_copy(rows, accum.at[eb.at[1]], sem)

        pltpu.sync_copy(zeros_hbm, accum.at[pl.ds(s * ZR, ZR)])
        plsc.subcore_barrier()

        edge_copy(0, eb0, sem_e0).start()
        w_copy(0, wb0, sem_e0).start()
        edge_copy(1, eb1, sem_e1).start()
        w_copy(1, wb1, sem_e1).start()
        edge_copy(0, eb0, sem_e0).wait()
        w_copy(0, wb0, sem_e0).wait()
        gather(eb0, rows0, sem_g0).start()
        edge_copy(1, eb1, sem_e1).wait()
        w_copy(1, wb1, sem_e1).wait()
        gather(eb1, rows1, sem_g1).start()

        def body(k, carry):
            i2 = 2 * k
            gather(eb0, rows0, sem_g0).wait()
            scale(wb0, rows0)
            scatter(eb0, rows0, sem_s0).start(add=True)
            gather(eb1, rows1, sem_g1).wait()
            scale(wb1, rows1)
            scatter(eb1, rows1, sem_s1).start(add=True)
            scatter(eb0, rows0, sem_s0).wait()
            edge_copy(i2 + 2, eb0, sem_e0).start()
            w_copy(i2 + 2, wb0, sem_e0).start()
            scatter(eb1, rows1, sem_s1).wait()
            edge_copy(i2 + 3, eb1, sem_e1).start()
            w_copy(i2 + 3, wb1, sem_e1).start()
            edge_copy(i2 + 2, eb0, sem_e0).wait()
            w_copy(i2 + 2, wb0, sem_e0).wait()
            gather(eb0, rows0, sem_g0).start()
            edge_copy(i2 + 3, eb1, sem_e1).wait()
            w_copy(i2 + 3, wb1, sem_e1).wait()
            gather(eb1, rows1, sem_g1).start()
            return carry

        lax.fori_loop(0, (NITER - 1) // 2, body, 0)

        gather(eb0, rows0, sem_g0).wait()
        scale(wb0, rows0)
        pltpu.sync_copy(rows0, accum.at[eb0.at[1]], add=True)
        gather(eb1, rows1, sem_g1).wait()

        plsc.subcore_barrier()
        pltpu.sync_copy(accum.at[pl.ds(s * ZR, ZR)],
                        out_hbm.at[c, pl.ds(s * ZR, ZR)])

    return spmm


_spmm128 = _build_spmm(NFEAT)


BM = 1000


def _dense1_body(x_ref, a_ref, b_ref, w1_ref, b1_ref, h_ref):
    t = x_ref[...] + a_ref[...] + b_ref[...]
    h = jnp.dot(t, w1_ref[...], preferred_element_type=jnp.float32)
    h_ref[...] = jnp.maximum(h + b1_ref[...], 0.0)


def _dense1(x, s1a, s1b, W1, b1):
    return pl.pallas_call(
        _dense1_body,
        grid=(N_NODES // BM,),
        in_specs=[
            pl.BlockSpec((BM, NFEAT), lambda i: (i, 0)),
            pl.BlockSpec((BM, NFEAT), lambda i: (i, 0)),
            pl.BlockSpec((BM, NFEAT), lambda i: (i, 0)),
            pl.BlockSpec((NFEAT, NFEAT), lambda i: (0, 0)),
            pl.BlockSpec((1, NFEAT), lambda i: (0, 0)),
        ],
        out_specs=pl.BlockSpec((BM, NFEAT), lambda i: (i, 0)),
        out_shape=jax.ShapeDtypeStruct((N_NODES, NFEAT), jnp.float32),
    )(x, s1a, s1b, W1, b1)


def _dense2_body(h_ref, a_ref, b_ref, w2_ref, b2_ref, o_ref):
    t = h_ref[...] + a_ref[...] + b_ref[...]
    z = jnp.dot(t, w2_ref[...], preferred_element_type=jnp.float32)
    z = z + b2_ref[...]
    m = jnp.max(z, axis=1, keepdims=True)
    zm = z - m
    o_ref[...] = zm - jnp.log(jnp.sum(jnp.exp(zm), axis=1, keepdims=True))


def _dense2(h, s2a, s2b, W2, b2):
    return pl.pallas_call(
        _dense2_body,
        grid=(N_NODES // BM,),
        in_specs=[
            pl.BlockSpec((BM, NFEAT), lambda i: (i, 0)),
            pl.BlockSpec((BM, NFEAT), lambda i: (i, 0)),
            pl.BlockSpec((BM, NFEAT), lambda i: (i, 0)),
            pl.BlockSpec((NFEAT, NCLASS), lambda i: (0, 0)),
            pl.BlockSpec((1, NCLASS), lambda i: (0, 0)),
        ],
        out_specs=pl.BlockSpec((BM, NCLASS), lambda i: (i, 0)),
        out_shape=jax.ShapeDtypeStruct((N_NODES, NCLASS), jnp.float32),
    )(h, s2a, s2b, W2, b2)


def kernel(x, edge_index, edge_weight, W1, b1, W2, b2):
    dst = edge_index[0].reshape(NW, NITER, B)
    src = edge_index[1].reshape(NW, NITER, B)
    edges = jnp.stack([src, dst], axis=2)
    edges = jnp.pad(edges, ((0, 0), (0, NWPAD - NITER), (0, 0), (0, 0)))
    w3 = edge_weight.reshape(NW, NITER, B)
    w3 = jnp.pad(w3, ((0, 0), (0, NWPAD - NITER), (0, 0)))
    zeros128 = jnp.zeros((ZR, NFEAT), jnp.float32)

    s1 = _spmm128(edges, w3, x, zeros128)
    h = _dense1(x, s1[0, :N_NODES], s1[1, :N_NODES], W1, b1.reshape(1, -1))
    s2 = _spmm128(edges, w3, h, zeros128)
    return _dense2(h, s2[0, :N_NODES], s2[1, :N_NODES], W2, b2.reshape(1, -1))


# speedup vs baseline: 7.9655x; 1.4139x over previous
import functools

import jax
import jax.numpy as jnp
from jax import lax
from jax.experimental import pallas as pl
from jax.experimental.pallas import tpu as pltpu
from jax.experimental.pallas import tpu_sc as plsc

N_NODES = 10000
N_EDGES = 320000
NFEAT = 128
NCLASS = 40

NC = 2
NS = 16
NW = NC * NS
EPT = N_EDGES // NW
B = 80
NITER = EPT // B
NPAD = 10240
ZR = NPAD // NS

_GDN = lax.GatherDimensionNumbers(
    offset_dims=(), collapsed_slice_dims=(0,), start_index_map=(0,))


def _build_spmm(D):
    mesh = plsc.VectorSubcoreMesh(core_axis_name="c", subcore_axis_name="s")

    def scale(wb, rows):
        for g in range(B // 16):
            w16 = wb[pl.ds(g * 16, 16)]
            for j in range(16):
                wv = lax.gather(
                    w16, jnp.full((16, 1), j, jnp.int32), _GDN,
                    slice_sizes=(1,),
                    mode=lax.GatherScatterMode.PROMISE_IN_BOUNDS)
                e = g * 16 + j
                for d in range(D // 16):
                    sl = (e, pl.ds(d * 16, 16))
                    rows[sl] = rows[sl] * wv

    @functools.partial(
        pl.kernel,
        out_type=jax.ShapeDtypeStruct((NC, NPAD, D), jnp.float32),
        mesh=mesh,
        scratch_types=[
            pltpu.VMEM((2, B), jnp.int32),
            pltpu.VMEM((2, B), jnp.int32),
            pltpu.VMEM((B,), jnp.float32),
            pltpu.VMEM((B,), jnp.float32),
            pltpu.VMEM((B, D), jnp.float32),
            pltpu.VMEM((B, D), jnp.float32),
            pltpu.VMEM_SHARED((NPAD, D), jnp.float32),
            pltpu.SemaphoreType.DMA,
            pltpu.SemaphoreType.DMA,
            pltpu.SemaphoreType.DMA,
            pltpu.SemaphoreType.DMA,
            pltpu.SemaphoreType.DMA,
            pltpu.SemaphoreType.DMA,
        ],
    )
    def spmm(edges_hbm, w_hbm, table_hbm, zeros_hbm, out_hbm,
             eb0, eb1, wb0, wb1, rows0, rows1, accum,
             sem_e0, sem_e1, sem_g0, sem_g1, sem_s0, sem_s1):
        c = lax.axis_index("c")
        s = lax.axis_index("s")
        wid = c * NS + s

        def edge_copy(i, eb, sem):
            return pltpu.make_async_copy(edges_hbm.at[wid, i], eb, sem)

        def w_copy(i, wb, sem):
            return pltpu.make_async_copy(w_hbm.at[wid, i], wb, sem)

        def gather(eb, rows, sem):
            return pltpu.make_async_copy(table_hbm.at[eb.at[0]], rows, sem)

        def scatter(eb, rows, sem):
            return pltpu.make_async_copy(rows, accum.at[eb.at[1]], sem)

        pltpu.sync_copy(zeros_hbm, accum.at[pl.ds(s * ZR, ZR)])
        plsc.subcore_barrier()

        edge_copy(0, eb0, sem_e0).start()
        w_copy(0, wb0, sem_e0).start()
        edge_copy(1, eb1, sem_e1).start()
        w_copy(1, wb1, sem_e1).start()
        edge_copy(0, eb0, sem_e0).wait()
        w_copy(0, wb0, sem_e0).wait()
        gather(eb0, rows0, sem_g0).start()
        edge_copy(1, eb1, sem_e1).wait()
        w_copy(1, wb1, sem_e1).wait()
        gather(eb1, rows1, sem_g1).start()

        def body(k, carry):
            i2 = 2 * k
            gather(eb0, rows0, sem_g0).wait()
            scale(wb0, rows0)
            scatter(eb0, rows0, sem_s0).start(add=True)
            gather(eb1, rows1, sem_g1).wait()
            scale(wb1, rows1)
            scatter(eb1, rows1, sem_s1).start(add=True)
            ip0 = jnp.minimum(i2 + 2, NITER - 1)
            ip1 = jnp.minimum(i2 + 3, NITER - 1)
            scatter(eb0, rows0, sem_s0).wait()
            edge_copy(ip0, eb0, sem_e0).start()
            w_copy(ip0, wb0, sem_e0).start()
            scatter(eb1, rows1, sem_s1).wait()
            edge_copy(ip1, eb1, sem_e1).start()
            w_copy(ip1, wb1, sem_e1).start()
            edge_copy(ip0, eb0, sem_e0).wait()
            w_copy(ip0, wb0, sem_e0).wait()
            gather(eb0, rows0, sem_g0).start()
            edge_copy(ip1, eb1, sem_e1).wait()
            w_copy(ip1, wb1, sem_e1).wait()
            gather(eb1, rows1, sem_g1).start()
            return carry

        lax.fori_loop(0, (NITER - 1) // 2, body, 0)

        gather(eb0, rows0, sem_g0).wait()
        scale(wb0, rows0)
        pltpu.sync_copy(rows0, accum.at[eb0.at[1]], add=True)
        gather(eb1, rows1, sem_g1).wait()

        plsc.subcore_barrier()
        pltpu.sync_copy(accum.at[pl.ds(s * ZR, ZR)],
                        out_hbm.at[c, pl.ds(s * ZR, ZR)])

    return spmm


_spmm128 = _build_spmm(NFEAT)


BM = 1000


def _dense1_body(x_ref, s_ref, w1_ref, b1_ref, h_ref):
    t = x_ref[...] + s_ref[0] + s_ref[1]
    h = jnp.dot(t, w1_ref[...], preferred_element_type=jnp.float32)
    h_ref[...] = jnp.maximum(h + b1_ref[...], 0.0)


def _dense1(x, s1, W1, b1):
    return pl.pallas_call(
        _dense1_body,
        grid=(N_NODES // BM,),
        in_specs=[
            pl.BlockSpec((BM, NFEAT), lambda i: (i, 0)),
            pl.BlockSpec((2, BM, NFEAT), lambda i: (0, i, 0)),
            pl.BlockSpec((NFEAT, NFEAT), lambda i: (0, 0)),
            pl.BlockSpec((1, NFEAT), lambda i: (0, 0)),
        ],
        out_specs=pl.BlockSpec((BM, NFEAT), lambda i: (i, 0)),
        out_shape=jax.ShapeDtypeStruct((N_NODES, NFEAT), jnp.float32),
    )(x, s1, W1, b1)


def _dense2_body(h_ref, s_ref, w2_ref, b2_ref, o_ref):
    t = h_ref[...] + s_ref[0] + s_ref[1]
    z = jnp.dot(t, w2_ref[...], preferred_element_type=jnp.float32)
    z = z + b2_ref[...]
    m = jnp.max(z, axis=1, keepdims=True)
    zm = z - m
    o_ref[...] = zm - jnp.log(jnp.sum(jnp.exp(zm), axis=1, keepdims=True))


def _dense2(h, s2, W2, b2):
    return pl.pallas_call(
        _dense2_body,
        grid=(N_NODES // BM,),
        in_specs=[
            pl.BlockSpec((BM, NFEAT), lambda i: (i, 0)),
            pl.BlockSpec((2, BM, NFEAT), lambda i: (0, i, 0)),
            pl.BlockSpec((NFEAT, NCLASS), lambda i: (0, 0)),
            pl.BlockSpec((1, NCLASS), lambda i: (0, 0)),
        ],
        out_specs=pl.BlockSpec((BM, NCLASS), lambda i: (i, 0)),
        out_shape=jax.ShapeDtypeStruct((N_NODES, NCLASS), jnp.float32),
    )(h, s2, W2, b2)


def kernel(x, edge_index, edge_weight, W1, b1, W2, b2):
    dst = edge_index[0].reshape(NW, NITER, B)
    src = edge_index[1].reshape(NW, NITER, B)
    edges = jnp.stack([src, dst], axis=2)
    w3 = edge_weight.reshape(NW, NITER, B)
    zeros128 = jnp.zeros((ZR, NFEAT), jnp.float32)

    s1 = _spmm128(edges, w3, x, zeros128)
    h = _dense1(x, s1, W1, b1.reshape(1, -1))
    s2 = _spmm128(edges, w3, h, zeros128)
    return _dense2(h, s2, W2, b2.reshape(1, -1))


# speedup vs baseline: 9.1420x; 1.1477x over previous
import functools

import jax
import jax.numpy as jnp
from jax import lax
from jax.experimental import pallas as pl
from jax.experimental.pallas import tpu as pltpu
from jax.experimental.pallas import tpu_sc as plsc

N_NODES = 10000
N_EDGES = 320000
NFEAT = 128
NCLASS = 40

NC = 2
NS = 16
NW = NC * NS
EPT = N_EDGES // NW
B = 80
NITER = EPT // B
NPAD = 10240
ZR = NPAD // NS

_GDN = lax.GatherDimensionNumbers(
    offset_dims=(), collapsed_slice_dims=(0,), start_index_map=(0,))


def _build_spmm(D):
    mesh = plsc.VectorSubcoreMesh(core_axis_name="c", subcore_axis_name="s")

    def scale(wb, rows):
        for g in range(B // 16):
            w16 = wb[pl.ds(g * 16, 16)]
            for j in range(16):
                wv = lax.gather(
                    w16, jnp.full((16, 1), j, jnp.int32), _GDN,
                    slice_sizes=(1,),
                    mode=lax.GatherScatterMode.PROMISE_IN_BOUNDS)
                e = g * 16 + j
                for d in range(D // 16):
                    sl = (e, pl.ds(d * 16, 16))
                    rows[sl] = rows[sl] * wv

    @functools.partial(
        pl.kernel,
        out_type=jax.ShapeDtypeStruct((NC, NPAD, D), jnp.float32),
        mesh=mesh,
        scratch_types=[
            pltpu.VMEM((NITER * 2 * B,), jnp.int32),
            pltpu.VMEM((B,), jnp.float32),
            pltpu.VMEM((B,), jnp.float32),
            pltpu.VMEM((B, D), jnp.float32),
            pltpu.VMEM((B, D), jnp.float32),
            pltpu.VMEM_SHARED((NPAD, D), jnp.float32),
            pltpu.SemaphoreType.DMA,
            pltpu.SemaphoreType.DMA,
            pltpu.SemaphoreType.DMA,
            pltpu.SemaphoreType.DMA,
            pltpu.SemaphoreType.DMA,
            pltpu.SemaphoreType.DMA,
        ],
    )
    def spmm(edges_hbm, w_hbm, table_hbm, zeros_hbm, out_hbm,
             eb_all, wb0, wb1, rows0, rows1, accum,
             sem_e0, sem_e1, sem_g0, sem_g1, sem_s0, sem_s1):
        c = lax.axis_index("c")
        s = lax.axis_index("s")
        wid = c * NS + s

        def w_copy(i, wb, sem):
            return pltpu.make_async_copy(w_hbm.at[wid, i], wb, sem)

        def gather(i, rows, sem):
            idx = eb_all.at[pl.ds(i * 2 * B, B)]
            return pltpu.make_async_copy(table_hbm.at[idx], rows, sem)

        def scatter(i, rows, sem):
            idx = eb_all.at[pl.ds(i * 2 * B + B, B)]
            return pltpu.make_async_copy(rows, accum.at[idx], sem)

        pltpu.sync_copy(edges_hbm.at[wid], eb_all)
        pltpu.sync_copy(zeros_hbm, accum.at[pl.ds(s * ZR, ZR)])
        plsc.subcore_barrier()

        w_copy(0, wb0, sem_e0).start()
        w_copy(1, wb1, sem_e1).start()
        gather(0, rows0, sem_g0).start()
        gather(1, rows1, sem_g1).start()
        w_copy(0, wb0, sem_e0).wait()
        w_copy(1, wb1, sem_e1).wait()

        def body(k, carry):
            i2 = 2 * k
            gather(i2, rows0, sem_g0).wait()
            scale(wb0, rows0)
            scatter(i2, rows0, sem_s0).start(add=True)
            gather(i2 + 1, rows1, sem_g1).wait()
            scale(wb1, rows1)
            scatter(i2 + 1, rows1, sem_s1).start(add=True)
            ip0 = jnp.minimum(i2 + 2, NITER - 1)
            ip1 = jnp.minimum(i2 + 3, NITER - 1)
            scatter(i2, rows0, sem_s0).wait()
            gather(ip0, rows0, sem_g0).start()
            w_copy(ip0, wb0, sem_e0).start()
            scatter(i2 + 1, rows1, sem_s1).wait()
            gather(ip1, rows1, sem_g1).start()
            w_copy(ip1, wb1, sem_e1).start()
            w_copy(ip0, wb0, sem_e0).wait()
            w_copy(ip1, wb1, sem_e1).wait()
            return carry

        lax.fori_loop(0, (NITER - 1) // 2, body, 0)

        gather(NITER - 1, rows0, sem_g0).wait()
        scale(wb0, rows0)
        pltpu.sync_copy(
            rows0, accum.at[eb_all.at[pl.ds((NITER - 1) * 2 * B + B, B)]],
            add=True)
        gather(NITER - 1, rows1, sem_g1).wait()

        plsc.subcore_barrier()
        pltpu.sync_copy(accum.at[pl.ds(s * ZR, ZR)],
                        out_hbm.at[c, pl.ds(s * ZR, ZR)])

    return spmm


_spmm128 = _build_spmm(NFEAT)


BM = 1000


def _dense1_body(x_ref, s_ref, w1_ref, b1_ref, h_ref):
    t = x_ref[...] + s_ref[0] + s_ref[1]
    h = jnp.dot(t, w1_ref[...], preferred_element_type=jnp.float32)
    h_ref[...] = jnp.maximum(h + b1_ref[...], 0.0)


def _dense1(x, s1, W1, b1):
    return pl.pallas_call(
        _dense1_body,
        grid=(N_NODES // BM,),
        in_specs=[
            pl.BlockSpec((BM, NFEAT), lambda i: (i, 0)),
            pl.BlockSpec((2, BM, NFEAT), lambda i: (0, i, 0)),
            pl.BlockSpec((NFEAT, NFEAT), lambda i: (0, 0)),
            pl.BlockSpec((1, NFEAT), lambda i: (0, 0)),
        ],
        out_specs=pl.BlockSpec((BM, NFEAT), lambda i: (i, 0)),
        out_shape=jax.ShapeDtypeStruct((N_NODES, NFEAT), jnp.float32),
    )(x, s1, W1, b1)


def _dense2_body(h_ref, s_ref, w2_ref, b2_ref, o_ref):
    t = h_ref[...] + s_ref[0] + s_ref[1]
    z = jnp.dot(t, w2_ref[...], preferred_element_type=jnp.float32)
    z = z + b2_ref[...]
    m = jnp.max(z, axis=1, keepdims=True)
    zm = z - m
    o_ref[...] = zm - jnp.log(jnp.sum(jnp.exp(zm), axis=1, keepdims=True))


def _dense2(h, s2, W2, b2):
    return pl.pallas_call(
        _dense2_body,
        grid=(N_NODES // BM,),
        in_specs=[
            pl.BlockSpec((BM, NFEAT), lambda i: (i, 0)),
            pl.BlockSpec((2, BM, NFEAT), lambda i: (0, i, 0)),
            pl.BlockSpec((NFEAT, NCLASS), lambda i: (0, 0)),
            pl.BlockSpec((1, NCLASS), lambda i: (0, 0)),
        ],
        out_specs=pl.BlockSpec((BM, NCLASS), lambda i: (i, 0)),
        out_shape=jax.ShapeDtypeStruct((N_NODES, NCLASS), jnp.float32),
    )(h, s2, W2, b2)


def kernel(x, edge_index, edge_weight, W1, b1, W2, b2):
    dst = edge_index[0].reshape(NW, NITER, B)
    src = edge_index[1].reshape(NW, NITER, B)
    edges = jnp.stack([src, dst], axis=2).reshape(NW, NITER * 2 * B)
    w3 = edge_weight.reshape(NW, NITER, B)
    zeros128 = jnp.zeros((ZR, NFEAT), jnp.float32)

    s1 = _spmm128(edges, w3, x, zeros128)
    h = _dense1(x, s1, W1, b1.reshape(1, -1))
    s2 = _spmm128(edges, w3, h, zeros128)
    return _dense2(h, s2, W2, b2.reshape(1, -1))


# speedup vs baseline: 9.2317x; 1.0098x over previous
import functools

import jax
import jax.numpy as jnp
from jax import lax
from jax.experimental import pallas as pl
from jax.experimental.pallas import tpu as pltpu
from jax.experimental.pallas import tpu_sc as plsc

N_NODES = 10000
N_EDGES = 320000
NFEAT = 128
NCLASS = 40

NC = 2
NS = 16
NW = NC * NS
EPT = N_EDGES // NW
B = 80
NITER = EPT // B
NPAD = 10240
ZR = NPAD // NS

_GDN = lax.GatherDimensionNumbers(
    offset_dims=(), collapsed_slice_dims=(0,), start_index_map=(0,))


def _build_spmm(D):
    mesh = plsc.VectorSubcoreMesh(core_axis_name="c", subcore_axis_name="s")

    def scale(wb, rows):
        for g in range(B // 16):
            w16 = wb[pl.ds(g * 16, 16)]
            for j in range(16):
                wv = lax.gather(
                    w16, jnp.full((16, 1), j, jnp.int32), _GDN,
                    slice_sizes=(1,),
                    mode=lax.GatherScatterMode.PROMISE_IN_BOUNDS)
                e = g * 16 + j
                for d in range(D // 16):
                    sl = (e, pl.ds(d * 16, 16))
                    rows[sl] = rows[sl] * wv

    @functools.partial(
        pl.kernel,
        out_type=jax.ShapeDtypeStruct((NC, NPAD, D), jnp.float32),
        mesh=mesh,
        scratch_types=[
            pltpu.VMEM((NITER * 2 * B,), jnp.int32),
            pltpu.VMEM((B,), jnp.float32),
            pltpu.VMEM((B,), jnp.float32),
            pltpu.VMEM((B, D), jnp.float32),
            pltpu.VMEM((B, D), jnp.float32),
            pltpu.VMEM_SHARED((NPAD, D), jnp.float32),
            pltpu.SemaphoreType.DMA,
            pltpu.SemaphoreType.DMA,
            pltpu.SemaphoreType.DMA,
            pltpu.SemaphoreType.DMA,
            pltpu.SemaphoreType.DMA,
            pltpu.SemaphoreType.DMA,
        ],
    )
    def spmm(edges_hbm, w_hbm, table_hbm, zeros_hbm, out_hbm,
             eb_all, wb0, wb1, rows0, rows1, accum,
             sem_e0, sem_e1, sem_g0, sem_g1, sem_s0, sem_s1):
        c = lax.axis_index("c")
        s = lax.axis_index("s")
        wid = c * NS + s

        def w_copy(i, wb, sem):
            return pltpu.make_async_copy(w_hbm.at[wid, i], wb, sem)

        def gather(i, rows, sem):
            idx = eb_all.at[pl.ds(i * 2 * B, B)]
            return pltpu.make_async_copy(table_hbm.at[idx], rows, sem)

        def scatter(i, rows, sem):
            idx = eb_all.at[pl.ds(i * 2 * B + B, B)]
            return pltpu.make_async_copy(rows, accum.at[idx], sem)

        eb_stage = pltpu.make_async_copy(edges_hbm.at[wid], eb_all, sem_s0)
        eb_stage.start()
        zclear = pltpu.make_async_copy(
            zeros_hbm, accum.at[pl.ds(s * ZR, ZR)], sem_s1)
        zclear.start()
        eb_stage.wait()
        w_copy(0, wb0, sem_e0).start()
        w_copy(1, wb1, sem_e1).start()
        gather(0, rows0, sem_g0).start()
        gather(1, rows1, sem_g1).start()
        zclear.wait()
        plsc.subcore_barrier()
        w_copy(0, wb0, sem_e0).wait()
        w_copy(1, wb1, sem_e1).wait()

        def body(k, carry):
            i2 = 2 * k
            gather(i2, rows0, sem_g0).wait()
            scale(wb0, rows0)
            scatter(i2, rows0, sem_s0).start(add=True)
            gather(i2 + 1, rows1, sem_g1).wait()
            scale(wb1, rows1)
            scatter(i2 + 1, rows1, sem_s1).start(add=True)
            ip0 = jnp.minimum(i2 + 2, NITER - 1)
            ip1 = jnp.minimum(i2 + 3, NITER - 1)
            scatter(i2, rows0, sem_s0).wait()
            gather(ip0, rows0, sem_g0).start()
            w_copy(ip0, wb0, sem_e0).start()
            scatter(i2 + 1, rows1, sem_s1).wait()
            gather(ip1, rows1, sem_g1).start()
            w_copy(ip1, wb1, sem_e1).start()
            w_copy(ip0, wb0, sem_e0).wait()
            w_copy(ip1, wb1, sem_e1).wait()
            return carry

        lax.fori_loop(0, (NITER - 1) // 2, body, 0)

        gather(NITER - 1, rows0, sem_g0).wait()
        scale(wb0, rows0)
        pltpu.sync_copy(
            rows0, accum.at[eb_all.at[pl.ds((NITER - 1) * 2 * B + B, B)]],
            add=True)
        gather(NITER - 1, rows1, sem_g1).wait()

        plsc.subcore_barrier()
        pltpu.sync_copy(accum.at[pl.ds(s * ZR, ZR)],
                        out_hbm.at[c, pl.ds(s * ZR, ZR)])

    return spmm


_spmm128 = _build_spmm(NFEAT)


BM = 1000


def _dense1_body(x_ref, s_ref, w1_ref, b1_ref, h_ref):
    t = x_ref[...] + s_ref[0] + s_ref[1]
    h = jnp.dot(t, w1_ref[...], preferred_element_type=jnp.float32)
    h_ref[...] = jnp.maximum(h + b1_ref[...], 0.0)


def _dense1(x, s1, W1, b1):
    return pl.pallas_call(
        _dense1_body,
        grid=(N_NODES // BM,),
        in_specs=[
            pl.BlockSpec((BM, NFEAT), lambda i: (i, 0)),
            pl.BlockSpec((2, BM, NFEAT), lambda i: (0, i, 0)),
            pl.BlockSpec((NFEAT, NFEAT), lambda i: (0, 0)),
            pl.BlockSpec((1, NFEAT), lambda i: (0, 0)),
        ],
        out_specs=pl.BlockSpec((BM, NFEAT), lambda i: (i, 0)),
        out_shape=jax.ShapeDtypeStruct((N_NODES, NFEAT), jnp.float32),
    )(x, s1, W1, b1)


def _dense2_body(h_ref, s_ref, w2_ref, b2_ref, o_ref):
    t = h_ref[...] + s_ref[0] + s_ref[1]
    z = jnp.dot(t, w2_ref[...], preferred_element_type=jnp.float32)
    z = z + b2_ref[...]
    m = jnp.max(z, axis=1, keepdims=True)
    zm = z - m
    o_ref[...] = zm - jnp.log(jnp.sum(jnp.exp(zm), axis=1, keepdims=True))


def _dense2(h, s2, W2, b2):
    return pl.pallas_call(
        _dense2_body,
        grid=(N_NODES // BM,),
        in_specs=[
            pl.BlockSpec((BM, NFEAT), lambda i: (i, 0)),
            pl.BlockSpec((2, BM, NFEAT), lambda i: (0, i, 0)),
            pl.BlockSpec((NFEAT, NCLASS), lambda i: (0, 0)),
            pl.BlockSpec((1, NCLASS), lambda i: (0, 0)),
        ],
        out_specs=pl.BlockSpec((BM, NCLASS), lambda i: (i, 0)),
        out_shape=jax.ShapeDtypeStruct((N_NODES, NCLASS), jnp.float32),
    )(h, s2, W2, b2)


def kernel(x, edge_index, edge_weight, W1, b1, W2, b2):
    dst = edge_index[0].reshape(NW, NITER, B)
    src = edge_index[1].reshape(NW, NITER, B)
    edges = jnp.stack([src, dst], axis=2).reshape(NW, NITER * 2 * B)
    w3 = edge_weight.reshape(NW, NITER, B)
    zeros128 = jnp.zeros((ZR, NFEAT), jnp.float32)

    s1 = _spmm128(edges, w3, x, zeros128)
    h = _dense1(x, s1, W1, b1.reshape(1, -1))
    s2 = _spmm128(edges, w3, h, zeros128)
    return _dense2(h, s2, W2, b2.reshape(1, -1))
